# baseline (speedup 1.0000x reference)
import dataclasses

import jax
import jax.numpy as jnp
import numpy as np
from jax import lax
from jax.experimental import pallas as pl
from jax.experimental.pallas import tpu as pltpu
from jax.experimental.pallas import tpu_sc as plsc

_N = 16384
_C = 10
_NB = 15
_L = 16
_NTILES = 32
_ROWS_PER_TILE = _N // _NTILES
_BLKS = _ROWS_PER_TILE // _L
_PROWS = 98

_INV_BIN_W = float(1.0 / 15)
_BOUNDS_NP = np.linspace(0.0, 1.0, _NB + 1).astype(np.float32)


_HALF = _ROWS_PER_TILE // 2
_HBLKS = _HALF // _L


def _sc_body(x_hbm, t_hbm, b_hbm, out_hbm, xv0, xv1, tv, bv, P, tm,
             sem0, sem1, semt):
    wid = lax.axis_index("c") * 16 + lax.axis_index("s")
    base = wid * _ROWS_PER_TILE
    base10 = base * _C
    cp0 = pltpu.async_copy(
        x_hbm.at[pl.ds(base10, _HALF * _C)], xv0, sem0)
    cp1 = pltpu.async_copy(
        x_hbm.at[pl.ds(base10 + _HALF * _C, _HALF * _C)], xv1, sem1)
    cpt = pltpu.async_copy(t_hbm.at[pl.ds(base, _ROWS_PER_TILE)], tv, semt)
    pltpu.sync_copy(b_hbm, bv)

    zf = jnp.zeros((_L,), jnp.float32)

    @pl.loop(0, _PROWS)
    def _zero(r):
        P.at[r][...] = zf

    lanes = lax.iota(jnp.int32, _L)
    lanes10 = lanes * _C
    ones = jnp.ones((_L,), jnp.float32)

    def _tree(vals, op):
        while len(vals) > 1:
            vals = [op(vals[i], vals[i + 1]) if i + 1 < len(vals) else vals[i]
                    for i in range(0, len(vals), 2)]
        return vals[0]

    def _process_half(buf):

        @pl.loop(0, _HBLKS, step=2)
        def _blk(j0):
            for u in range(2):
                r0 = (j0 + u) * _L
                idx0 = r0 * _C + lanes10
                xs = [plsc.load_gather(buf, [idx0 + k]) for k in range(_C)]
                m = _tree(xs, jnp.maximum)
                es = [jnp.exp(x - m) for x in xs]
                s = _tree(es, lambda a, b: a + b)
                e2 = _tree([e * e for e in es], lambda a, b: a + b)
                inv = 1.0 / s
                plsc.addupdate(P.at[96], e2 * inv * inv)

                for col, off in ((0, 0), (1, 48)):
                    cc = es[col] / s
                    q = cc / _INV_BIN_W
                    fb = jnp.minimum(q.astype(jnp.int32), 14)
                    g1 = plsc.load_gather(bv, [fb])
                    g2 = plsc.load_gather(bv, [fb + 1])
                    tb = (fb - (cc <= g1).astype(jnp.int32)
                          + (cc > g2).astype(jnp.int32))
                    plsc.addupdate_scatter(P, [fb + off, lanes], cc)
                    plsc.addupdate_scatter(P, [fb + (off + 16), lanes], ones)
                    plsc.addupdate_scatter(P, [tb + (off + 32), lanes], ones,
                                           mask=tb >= 0)

    cp0.wait()
    _process_half(xv0)
    cp1.wait()
    _process_half(xv1)

    cpt.wait()
    tm[...] = tv[pl.ds(0, _L)]

    @pl.loop(1, _BLKS)
    def _tmax(j):
        tm[...] = jnp.maximum(tm[...], tv[pl.ds(j * _L, _L)])

    P.at[97][...] = tm[...].astype(jnp.float32)
    pltpu.sync_copy(P, out_hbm.at[wid])


def _fin_body(p_ref, o_ref):
    p = p_ref[...]
    rowsum = jnp.sum(p, axis=(0, 2))
    n_oh = jnp.max(p[:, 97, :]) + 1.0
    a = (n_oh - 1.0) / n_oh
    c = 1.0 / n_oh
    S0 = rowsum[0:16]
    C0 = rowsum[16:32]
    O0 = rowsum[32:48]
    S1 = rowsum[48:64]
    C1 = rowsum[64:80]
    O1 = rowsum[80:96]
    t0 = jnp.where(O0 > 0, a * a * C0 - 2.0 * a * S0, 0.0)
    t1 = jnp.where(O1 > 0, c * c * C1 - 2.0 * c * S1, 0.0)
    total = rowsum[96] + jnp.sum(t0) + jnp.sum(t1)
    o_ref[...] = (total / np.float32(_N * _C)).reshape(1, 1)


def _sc_compiler_params():
    cp = pltpu.CompilerParams()
    if "needs_layout_passes" in pltpu.CompilerParams.__dataclass_fields__:
        cp = dataclasses.replace(cp, needs_layout_passes=False)
    return cp


def kernel(input, target):
    bounds = jnp.asarray(_BOUNDS_NP)
    sc_pass = pl.kernel(
        _sc_body,
        out_type=jax.ShapeDtypeStruct((_NTILES, _PROWS, _L), jnp.float32),
        mesh=plsc.VectorSubcoreMesh(core_axis_name="c", subcore_axis_name="s"),
        scratch_types=[
            pltpu.VMEM((_HALF * _C,), jnp.float32),
            pltpu.VMEM((_HALF * _C,), jnp.float32),
            pltpu.VMEM((_ROWS_PER_TILE,), jnp.int32),
            pltpu.VMEM((_NB + 1,), jnp.float32),
            pltpu.VMEM((_PROWS, _L), jnp.float32),
            pltpu.VMEM((_L,), jnp.int32),
            pltpu.SemaphoreType.DMA,
            pltpu.SemaphoreType.DMA,
            pltpu.SemaphoreType.DMA,
        ],
        compiler_params=_sc_compiler_params(),
    )
    partials = sc_pass(input.reshape(-1), target, bounds)
    out = pl.pallas_call(
        _fin_body,
        out_shape=jax.ShapeDtypeStruct((1, 1), jnp.float32),
    )(partials)
    return out[0, 0]


# speedup vs baseline: 1.4986x; 1.4986x over previous
import dataclasses

import jax
import jax.numpy as jnp
import numpy as np
from jax import lax
from jax.experimental import pallas as pl
from jax.experimental.pallas import tpu as pltpu
from jax.experimental.pallas import tpu_sc as plsc

_N = 16384
_C = 10
_NB = 15
_L = 16
_NTILES = 32
_ROWS_PER_TILE = _N // _NTILES
_BLKS = _ROWS_PER_TILE // _L
_PROWS = 98

_INV_BIN_W = float(1.0 / 15)
_BOUNDS_NP = np.linspace(0.0, 1.0, _NB + 1).astype(np.float32)


_HALF = _ROWS_PER_TILE // 2
_HBLKS = _HALF // _L


def _sc_body(x_hbm, t_hbm, b_hbm, out_hbm, xv0, xv1, tv, bv, P, tm,
             sem0, sem1, semt):
    wid = lax.axis_index("c") * 16 + lax.axis_index("s")
    base = wid * _ROWS_PER_TILE
    cp0 = pltpu.async_copy(x_hbm.at[:, pl.ds(base, _HALF)], xv0, sem0)
    cp1 = pltpu.async_copy(x_hbm.at[:, pl.ds(base + _HALF, _HALF)], xv1, sem1)
    cpt = pltpu.async_copy(t_hbm.at[pl.ds(base, _ROWS_PER_TILE)], tv, semt)
    pltpu.sync_copy(b_hbm, bv)

    zf = jnp.zeros((_L,), jnp.float32)

    @pl.loop(0, _PROWS)
    def _zero(r):
        P.at[r][...] = zf

    lanes = lax.iota(jnp.int32, _L)
    ones = jnp.ones((_L,), jnp.float32)

    def _tree(vals, op):
        while len(vals) > 1:
            vals = [op(vals[i], vals[i + 1]) if i + 1 < len(vals) else vals[i]
                    for i in range(0, len(vals), 2)]
        return vals[0]

    def _process_half(buf):

        @pl.loop(0, _HBLKS, step=2)
        def _blk(j0):
            for u in range(2):
                r0 = (j0 + u) * _L
                xs = [buf[k, pl.ds(r0, _L)] for k in range(_C)]
                m = _tree(xs, jnp.maximum)
                es = [jnp.exp(x - m) for x in xs]
                s = _tree(es, lambda a, b: a + b)
                e2 = _tree([e * e for e in es], lambda a, b: a + b)
                inv = 1.0 / s
                plsc.addupdate(P.at[96], e2 * inv * inv)

                for col, off in ((0, 0), (1, 48)):
                    cc = es[col] / s
                    q = cc / _INV_BIN_W
                    fb = jnp.minimum(q.astype(jnp.int32), 14)
                    g1 = plsc.load_gather(bv, [fb])
                    g2 = plsc.load_gather(bv, [fb + 1])
                    tb = (fb - (cc <= g1).astype(jnp.int32)
                          + (cc > g2).astype(jnp.int32))
                    plsc.addupdate_scatter(P, [fb + off, lanes], cc)
                    plsc.addupdate_scatter(P, [fb + (off + 16), lanes], ones)
                    plsc.addupdate_scatter(P, [tb + (off + 32), lanes], ones,
                                           mask=tb >= 0)

    cp0.wait()
    _process_half(xv0)
    cp1.wait()
    _process_half(xv1)

    cpt.wait()
    tm[...] = tv[pl.ds(0, _L)]

    @pl.loop(1, _BLKS)
    def _tmax(j):
        tm[...] = jnp.maximum(tm[...], tv[pl.ds(j * _L, _L)])

    P.at[97][...] = tm[...].astype(jnp.float32)
    pltpu.sync_copy(P, out_hbm.at[wid])


def _fin_body(p_ref, o_ref):
    p = p_ref[...]
    rowsum = jnp.sum(p, axis=(0, 2))
    n_oh = jnp.max(p[:, 97, :]) + 1.0
    a = (n_oh - 1.0) / n_oh
    c = 1.0 / n_oh
    S0 = rowsum[0:16]
    C0 = rowsum[16:32]
    O0 = rowsum[32:48]
    S1 = rowsum[48:64]
    C1 = rowsum[64:80]
    O1 = rowsum[80:96]
    t0 = jnp.where(O0 > 0, a * a * C0 - 2.0 * a * S0, 0.0)
    t1 = jnp.where(O1 > 0, c * c * C1 - 2.0 * c * S1, 0.0)
    total = rowsum[96] + jnp.sum(t0) + jnp.sum(t1)
    o_ref[...] = (total / np.float32(_N * _C)).reshape(1, 1)


def _sc_compiler_params():
    cp = pltpu.CompilerParams()
    if "needs_layout_passes" in pltpu.CompilerParams.__dataclass_fields__:
        cp = dataclasses.replace(cp, needs_layout_passes=False)
    return cp


def kernel(input, target):
    bounds = jnp.asarray(_BOUNDS_NP)
    sc_pass = pl.kernel(
        _sc_body,
        out_type=jax.ShapeDtypeStruct((_NTILES, _PROWS, _L), jnp.float32),
        mesh=plsc.VectorSubcoreMesh(core_axis_name="c", subcore_axis_name="s"),
        scratch_types=[
            pltpu.VMEM((_C, _HALF), jnp.float32),
            pltpu.VMEM((_C, _HALF), jnp.float32),
            pltpu.VMEM((_ROWS_PER_TILE,), jnp.int32),
            pltpu.VMEM((_NB + 1,), jnp.float32),
            pltpu.VMEM((_PROWS, _L), jnp.float32),
            pltpu.VMEM((_L,), jnp.int32),
            pltpu.SemaphoreType.DMA,
            pltpu.SemaphoreType.DMA,
            pltpu.SemaphoreType.DMA,
        ],
        compiler_params=_sc_compiler_params(),
    )
    partials = sc_pass(input.T, target, bounds)
    out = pl.pallas_call(
        _fin_body,
        out_shape=jax.ShapeDtypeStruct((1, 1), jnp.float32),
    )(partials)
    return out[0, 0]
